# baseline (speedup 1.0000x reference)
import functools
import math

import jax
import jax.numpy as jnp
from jax import lax
from jax.experimental import pallas as pl
from jax.experimental.pallas import tpu as pltpu
from jax.experimental.pallas import tpu_sc as plsc

D_MODEL = 64
SCALE = math.sqrt(D_MODEL)


@functools.lru_cache(maxsize=None)
def _make_sc_kernel(B: int, S: int, V: int, D: int):
    info = plsc.get_sparse_core_info()
    NC, NS, L = info.num_cores, info.num_subcores, info.num_lanes
    NW = NC * NS
    LB = 128
    n_blk = B // LB
    bpw = n_blk // NW
    n_grp = bpw * S
    assert B % (LB * NW) == 0 and D % L == 0 and n_grp % 2 == 0
    vp = D // L
    mesh = plsc.VectorSubcoreMesh(core_axis_name="c", subcore_axis_name="s")

    @functools.partial(
        pl.kernel,
        mesh=mesh,
        out_type=jax.ShapeDtypeStruct((S, D, B), jnp.float32),
        compiler_params=pltpu.CompilerParams(
            use_tc_tiling_on_sc=True, needs_layout_passes=False
        ),
        scratch_types=(
            [pltpu.VMEM((S, 4 * LB), jnp.int32)]
            + [pltpu.VMEM((LB,), jnp.int32) for _ in range(2)]
            + [pltpu.VMEM((LB,), jnp.int32) for _ in range(2)]
            + [pltpu.VMEM((LB, 2 * D + 9), jnp.float32) for _ in range(2)]
            + [pltpu.VMEM((D, LB), jnp.float32) for _ in range(2)]
            + [pltpu.SemaphoreType.DMA for _ in range(4)]
        ),
    )
    def k(xt_hbm, tab_hbm, out_hbm, xv_all, *scr):
        idx2 = scr[0:2]
        half = scr[2:4]
        rows = scr[4:6]
        outv = scr[6:8]
        gsem = scr[8:10]
        ssem = scr[10:12]
        wid = lax.axis_index("s") * NC + lax.axis_index("c")
        col0 = wid * (4 * LB)

        pltpu.sync_copy(xt_hbm.at[:, pl.ds(col0, 4 * LB)], xv_all)

        iota = lax.iota(jnp.int32, L)

        def prep(i, b):
            bo = i // S
            s = i - bo * S
            for kk in range(LB // L):
                v = xv_all[s, pl.ds(bo * LB + kk * L, L)]
                idx2[b][pl.ds(kk * L, L)] = v >> 1
                half[b][pl.ds(kk * L, L)] = (v & 1) << 6

        def start_gather(b):
            pltpu.async_copy(tab_hbm.at[idx2[b]], rows[b].at[:, pl.ds(0, 2 * D)], gsem[b])

        def start_store(i, b):
            bo = i // S
            s = i - bo * S
            pltpu.async_copy(
                outv[b], out_hbm.at[s, :, pl.ds(col0 + bo * LB, LB)], ssem[b]
            )

        prep(0, 0)
        start_gather(0)

        @pl.loop(0, n_grp, step=2)
        def _grp(g):
            for b in range(2):
                i = g + b
                nb = (b + 1) % 2

                @pl.when(i + 1 < n_grp)
                def _():
                    prep(i + 1, nb)
                    start_gather(nb)

                pltpu.make_async_copy(
                    tab_hbm.at[idx2[b]], rows[b].at[:, pl.ds(0, 2 * D)], gsem[b]
                ).wait()

                @pl.when(i >= 2)
                def _():
                    pltpu.make_async_copy(
                        outv[b], out_hbm.at[0, :, pl.ds(0, LB)], ssem[b]
                    ).wait()

                for bb in range(LB // L):
                    jrow = iota + bb * L
                    hv = half[b][pl.ds(bb * L, L)]

                    @plsc.parallel_loop(0, D, unroll=8, carry=hv)
                    def _shuf(dc, colv):
                        val = plsc.load_gather(rows[b], [jrow, colv])
                        outv[b][dc, pl.ds(bb * L, L)] = val * SCALE
                        return colv + 1

                start_store(i, b)

        for b in range(2):
            pltpu.make_async_copy(
                outv[b], out_hbm.at[0, :, pl.ds(0, LB)], ssem[b]
            ).wait()

    return k


def kernel(x, table):
    B, S = x.shape
    V, D = table.shape
    xt = x.T
    tab2 = table.reshape(V // 2, 2 * D)
    outp = _make_sc_kernel(B, S, V, D)(xt, tab2)
    return outp.transpose(2, 0, 1)


# speedup vs baseline: 1.3917x; 1.3917x over previous
import functools
import math

import jax
import jax.numpy as jnp
from jax import lax
from jax.experimental import pallas as pl
from jax.experimental.pallas import tpu as pltpu
from jax.experimental.pallas import tpu_sc as plsc

D_MODEL = 64
SCALE = math.sqrt(D_MODEL)


@functools.lru_cache(maxsize=None)
def _make_pre_kernel(B: int, S: int):
    info = plsc.get_sparse_core_info()
    NC, NS = info.num_cores, info.num_subcores
    NW = NC * NS
    SB = (S + 7) // 8
    NBL = B // 128
    n_tiles = SB * NBL
    assert n_tiles % NW == 0
    tpw = n_tiles // NW
    mesh = plsc.VectorSubcoreMesh(core_axis_name="c", subcore_axis_name="s")

    @functools.partial(
        pl.kernel,
        mesh=mesh,
        out_type=jax.ShapeDtypeStruct((S, NBL, 128), jnp.int32),
        compiler_params=pltpu.CompilerParams(
            use_tc_tiling_on_sc=True, needs_layout_passes=False
        ),
        scratch_types=(
            [pltpu.VMEM((S, 1, 128), jnp.int32) for _ in range(2)]
            + [pltpu.SemaphoreType.DMA for _ in range(4)]
        ),
    )
    def k(xt_hbm, out_hbm, *scr):
        buf = scr[0:2]
        gsem = scr[2:4]
        ssem = scr[4:6]
        wid = lax.axis_index("s") * NC + lax.axis_index("c")
        spw = NBL // NW
        t0 = wid * spw

        def start_load(t, b):
            pltpu.async_copy(
                xt_hbm.at[:, pl.ds(t * 128, 128)], buf[b].at[:, 0, :], gsem[b]
            )

        def wait_load(t, b):
            pltpu.make_async_copy(
                xt_hbm.at[:, pl.ds(t * 128, 128)], buf[b].at[:, 0, :], gsem[b]
            ).wait()

        def start_store(t, b):
            pltpu.async_copy(buf[b], out_hbm.at[:, pl.ds(t, 1), :], ssem[b])

        def wait_store(t, b):
            pltpu.make_async_copy(
                buf[b], out_hbm.at[:, pl.ds(t, 1), :], ssem[b]
            ).wait()

        start_load(t0, 0)
        for ti in range(spw):
            b = ti % 2
            nb = (ti + 1) % 2
            if ti + 1 < spw:
                if ti + 1 >= 2:
                    wait_store(t0 + ti - 1, nb)
                start_load(t0 + ti + 1, nb)
            wait_load(t0 + ti, b)
            start_store(t0 + ti, b)
        wait_store(t0 + spw - 2, 0)
        wait_store(t0 + spw - 1, 1)

    return k


@functools.lru_cache(maxsize=None)
def _make_main_kernel(B: int, S: int, V: int, D: int):
    info = plsc.get_sparse_core_info()
    NC, NS, L = info.num_cores, info.num_subcores, info.num_lanes
    NW = NC * NS
    LB = 128
    bpw = B // (LB * NW)
    n_grp = S * bpw
    SB = (S + 7) // 8
    assert B % (LB * NW) == 0 and D % L == 0 and n_grp % 2 == 0
    mesh = plsc.VectorSubcoreMesh(core_axis_name="c", subcore_axis_name="s")

    @functools.partial(
        pl.kernel,
        mesh=mesh,
        out_type=jax.ShapeDtypeStruct((B, SB, 8, 2 * D), jnp.float32),
        compiler_params=pltpu.CompilerParams(use_tc_tiling_on_sc=False),
        scratch_types=(
            [pltpu.VMEM((S, bpw, LB), jnp.int32)]
            + [pltpu.VMEM((LB, D), jnp.float32) for _ in range(2)]
            + [pltpu.VMEM((LB, 1, 1, D), jnp.float32) for _ in range(2)]
            + [pltpu.SemaphoreType.DMA for _ in range(4)]
        ),
    )
    def k(xl_hbm, tab_hbm, out_hbm, xv_all, *scr):
        rows = scr[0:2]
        outr = scr[2:4]
        gsem = scr[4:6]
        ssem = scr[6:8]
        wid = lax.axis_index("s") * NC + lax.axis_index("c")
        bb0 = wid * bpw

        pltpu.sync_copy(xl_hbm.at[:, pl.ds(bb0, bpw), :], xv_all)

        def coords(i):
            bo = i // S
            s = i - bo * S
            return bo, s

        def start_gather(i, b):
            bo, s = coords(i)
            pltpu.async_copy(tab_hbm.at[xv_all.at[s, bo]], rows[b], gsem[b])

        def out_slice(i):
            bo, s = coords(i)
            so = s // 8
            ss = s - so * 8
            return out_hbm.at[
                pl.ds((bb0 + bo) * LB, LB),
                pl.ds(so, 1),
                pl.ds(ss, 1),
                pl.ds(0, D),
            ]

        def wait_store(i, b):
            pltpu.make_async_copy(outr[b], out_slice(i), ssem[b]).wait()

        start_gather(0, 0)

        @pl.loop(0, n_grp, step=2)
        def _grp(g):
            for b in range(2):
                i = g + b
                nb = (b + 1) % 2

                @pl.when(i + 1 < n_grp)
                def _():
                    start_gather(i + 1, nb)

                pltpu.make_async_copy(
                    tab_hbm.at[xv_all.at[0, 0]], rows[b], gsem[b]
                ).wait()

                @pl.when(i >= 2)
                def _():
                    wait_store(i - 2, b)

                @plsc.parallel_loop(0, LB, unroll=2)
                def _scale(r):
                    for kk in range(D // L):
                        sl = pl.ds(kk * L, L)
                        outr[b][r, 0, 0, sl] = rows[b][r, sl] * SCALE

                pltpu.async_copy(outr[b], out_slice(i), ssem[b])

        wait_store(n_grp - 2, 0)
        wait_store(n_grp - 1, 1)

    return k


def kernel(x, table):
    B, S = x.shape
    V, D = table.shape
    SB = (S + 7) // 8
    xl = _make_pre_kernel(B, S)(x.T)
    out5 = _make_main_kernel(B, S, V, D)(xl, table)
    out = out5.reshape(B, SB * 8, 2 * D)[:, :S, :D]
    return out
